# Initial kernel scaffold; baseline (speedup 1.0000x reference)
import jax
import jax.numpy as jnp
from jax.experimental import pallas as pl


def kernel(point_cloud, feat, params):
    raise NotImplementedError("write your pallas kernel here")


# speedup vs baseline: 2.2839x; 2.2839x over previous
import functools

import jax
import jax.numpy as jnp
from jax import lax
from jax.experimental import pallas as pl
from jax.experimental.pallas import tpu as pltpu
from jax.experimental.pallas import tpu_sc as plsc

_K = 8
_RB = 128
_RBM = 1024


def _knn_body(pcb_ref, pca_ref, pct_ref, cov_ref, sid1_ref, gidx_ref, w_ref):
    bi = pl.program_id(0)
    n = pca_ref.shape[1]
    pcb = pcb_ref[0]
    pca = pca_ref[0]
    pat = pct_ref[0]

    inner = -2.0 * lax.dot_general(pcb, pca, (((1,), (1,)), ((), ())),
                                   preferred_element_type=jnp.float32)
    xxb = jnp.sum(pcb * pcb, axis=1, keepdims=True)
    xxa = (pat[0:1, :] * pat[0:1, :] + pat[1:2, :] * pat[1:2, :]
           + pat[2:3, :] * pat[2:3, :])
    pd = (-xxb) - inner - xxa

    iota = lax.broadcasted_iota(jnp.int32, (_RB, n), 1)
    big = jnp.float32(1e30)

    msk = jnp.zeros((_RB, n), jnp.float32)
    near_idx = []
    nfx = []
    for _ in range(_K):
        pm = pd - msk
        v = jnp.max(pm, axis=1, keepdims=True)
        idx = jnp.min(jnp.where(pm == v, iota, n), axis=1, keepdims=True)
        oh = (iota == idx)
        msk = msk + oh.astype(jnp.float32) * big
        near_idx.append(idx)
        nfx.append(jnp.dot(oh.astype(jnp.float32), pca,
                           preferred_element_type=jnp.float32,
                           precision=lax.Precision.HIGHEST))

    mean = (nfx[0] + nfx[1] + nfx[2] + nfx[3]
            + nfx[4] + nfx[5] + nfx[6] + nfx[7]) * jnp.float32(0.125)
    ckb = [(fx - mean).astype(jnp.bfloat16).astype(jnp.float32) for fx in nfx]
    cov_cols = []
    for a in range(3):
        for bcol in range(3):
            acc = ckb[0][:, a:a + 1] * ckb[0][:, bcol:bcol + 1]
            for k in range(1, _K):
                acc = acc + ckb[k][:, a:a + 1] * ckb[k][:, bcol:bcol + 1]
            cov_cols.append(acc)
    cov_ref[0] = jnp.concatenate(cov_cols, axis=1)

    sid1_ref[0] = jnp.concatenate(near_idx, axis=1) + bi * n

    msk2 = jnp.zeros((_RB, n), jnp.float32)
    iota3 = lax.broadcasted_iota(jnp.int32, (_RB, 3), 1)
    dis_l, gidx_l, wcos_l = [], [], []
    for _ in range(_K):
        pm = pd + msk2
        v = jnp.min(pm, axis=1, keepdims=True)
        idx = jnp.min(jnp.where(pm == v, iota, n), axis=1, keepdims=True)
        oh = (iota == idx)
        msk2 = msk2 + oh.astype(jnp.float32) * big
        fx = jnp.dot(oh.astype(jnp.float32), pca,
                     preferred_element_type=jnp.float32,
                     precision=lax.Precision.HIGHEST)
        d = fx - pcb
        dis = jnp.maximum(jnp.sqrt(jnp.sum(d * d, axis=1, keepdims=True)),
                          jnp.float32(1e-16))
        pcos = jnp.cos(d / dis) ** 2
        bid = (d > 0).astype(jnp.int32) + 2 * iota3
        gidx_l.append((idx + bi * n) * 6 + bid)
        dis_l.append(dis)
        wcos_l.append(pcos)

    pdr = jnp.concatenate(dis_l, axis=1)
    p_r = jnp.max(pdr, axis=1, keepdims=True) * jnp.float32(1.1)
    p_d = (p_r - pdr) ** 2
    wnorm = p_d / (jnp.sum(p_d, axis=1, keepdims=True) + jnp.float32(1e-16))

    w_cols = [wnorm[:, k:k + 1] * wcos_l[k] for k in range(_K)]
    zf = jnp.zeros((_RB, 8), jnp.float32)
    zi = jnp.zeros((_RB, 8), jnp.int32)
    w_ref[0] = jnp.concatenate(w_cols + [zf], axis=1)
    gidx_ref[0] = jnp.concatenate(gidx_l + [zi], axis=1)


def _knn_call(pc):
    b, n, _ = pc.shape
    grid = (b, n // _RB)
    return pl.pallas_call(
        _knn_body,
        grid=grid,
        in_specs=[
            pl.BlockSpec((1, _RB, 3), lambda bi, ri: (bi, ri, 0)),
            pl.BlockSpec((1, n, 3), lambda bi, ri: (bi, 0, 0)),
            pl.BlockSpec((1, 3, n), lambda bi, ri: (bi, 0, 0)),
        ],
        out_specs=[
            pl.BlockSpec((1, _RB, 9), lambda bi, ri: (bi, ri, 0)),
            pl.BlockSpec((1, _RB, _K), lambda bi, ri: (bi, ri, 0)),
            pl.BlockSpec((1, _RB, 32), lambda bi, ri: (bi, ri, 0)),
            pl.BlockSpec((1, _RB, 32), lambda bi, ri: (bi, ri, 0)),
        ],
        out_shape=[
            jax.ShapeDtypeStruct((b, n, 9), jnp.float32),
            jax.ShapeDtypeStruct((b, n, _K), jnp.int32),
            jax.ShapeDtypeStruct((b, n, 32), jnp.int32),
            jax.ShapeDtypeStruct((b, n, 32), jnp.float32),
        ],
    )(pc, pc, pc.transpose(0, 2, 1))


def _dense(row_ins, fulls, out_dims, stats_dims, body, rbm=_RBM):
    m = row_ins[0].shape[0]
    nb = m // rbm
    nri, nfu, nod = len(row_ins), len(fulls), len(out_dims)
    n_stats = len(stats_dims)

    def kern(*refs):
        i = pl.program_id(0)
        rvals = [refs[j][...] for j in range(nri)]
        fvals = [refs[nri + j][...] for j in range(nfu)]
        outs, stats = body(rvals, fvals)
        for j in range(nod):
            refs[nri + nfu + j][...] = outs[j]
        for j in range(n_stats):
            y = stats[j]
            c = y.shape[1]
            contrib = jnp.concatenate(
                [jnp.sum(y, axis=0, keepdims=True),
                 jnp.sum(y * y, axis=0, keepdims=True),
                 jnp.zeros((6, c), jnp.float32)], axis=0)
            ref = refs[nri + nfu + nod + j]

            @pl.when(i == 0)
            def _():
                ref[...] = contrib

            @pl.when(i > 0)
            def _():
                ref[...] += contrib

    in_specs = (
        [pl.BlockSpec((rbm, t.shape[1]), lambda i: (i, 0)) for t in row_ins] +
        [pl.BlockSpec(t.shape, (lambda nd: (lambda i: (0,) * nd))(t.ndim))
         for t in fulls])
    out_specs = (
        [pl.BlockSpec((rbm, c), lambda i: (i, 0)) for c in out_dims] +
        [pl.BlockSpec((8, c), lambda i: (0, 0)) for c in stats_dims])
    out_shape = (
        [jax.ShapeDtypeStruct((m, c), jnp.float32) for c in out_dims] +
        [jax.ShapeDtypeStruct((8, c), jnp.float32) for c in stats_dims])
    res = pl.pallas_call(
        kern, grid=(nb,), in_specs=in_specs, out_specs=out_specs,
        out_shape=out_shape,
    )(*row_ins, *fulls)
    return res


def _mm(x, w):
    return jnp.dot(x, w, preferred_element_type=jnp.float32)


def _leaky(x):
    return jnp.where(x >= 0, x, jnp.float32(0.2) * x)


def _aff_leaky(y, aff):
    return _leaky(y * aff[0:1, :] + aff[1:2, :])


def _aff_sig(y, aff):
    return jax.nn.sigmoid(y * aff[0:1, :] + aff[1:2, :])


def _affine_from_stats(stats, g, be, mtot):
    mean = stats[0] / mtot
    var = stats[1] / mtot - mean * mean
    sc = g / jnp.sqrt(var + 1e-5)
    sh = be - mean * sc
    return jnp.concatenate([sc[None, :], sh[None, :],
                            jnp.zeros((6, sc.shape[0]), jnp.float32)], axis=0)


def _brow(b):
    return jnp.concatenate([b[None, :], jnp.zeros((7, b.shape[0]),
                                                  jnp.float32)], axis=0)


def _sc_gather(table, idx):
    e = idx.shape[0]
    v, c = table.shape
    info = plsc.get_sparse_core_info()
    nw = info.num_cores * info.num_subcores
    epw = e // nw
    nchunk = epw // 128
    mesh = plsc.VectorSubcoreMesh(core_axis_name="c", subcore_axis_name="s")

    @functools.partial(
        pl.kernel, mesh=mesh,
        out_type=jax.ShapeDtypeStruct((e, c), jnp.float32),
        scratch_types=[
            pltpu.VMEM((128,), jnp.int32),
            pltpu.VMEM((128, c), jnp.float32),
            pltpu.SemaphoreType.DMA,
        ])
    def k(table_hbm, idx_hbm, out_hbm, idx_v, rows_v, sem):
        wid = lax.axis_index("s") * info.num_cores + lax.axis_index("c")
        base_e = wid * epw

        def chunk_body(ci, _):
            ebase = base_e + ci * 128
            pltpu.sync_copy(idx_hbm.at[pl.ds(ebase, 128)], idx_v)
            pltpu.async_copy(table_hbm.at[idx_v], rows_v, sem).wait()
            pltpu.sync_copy(rows_v, out_hbm.at[pl.ds(ebase, 128)])
            return 0

        lax.fori_loop(0, nchunk, chunk_body, 0)

    return k(table, idx)


def _pp_call(xf, xs3, wt, wb, bias):
    m, c = xf.shape
    cs = xs3.shape[2]
    co = wt.shape[1]
    rb = 128

    def body(xf_ref, xs_ref, wt_ref, wb_ref, b_ref, o_ref):
        xt = xf_ref[...]
        base = jnp.dot(xt, wt_ref[...],
                       preferred_element_type=jnp.float32) + b_ref[0:1, :]
        acc = None
        for k in range(_K):
            diff = xs_ref[:, k, :c] - xt
            h = jnp.maximum(
                base + jnp.dot(diff, wb_ref[...],
                               preferred_element_type=jnp.float32), 0.0)
            acc = h if acc is None else jnp.maximum(acc, h)
        o_ref[...] = acc

    return pl.pallas_call(
        body, grid=(m // rb,),
        in_specs=[pl.BlockSpec((rb, c), lambda i: (i, 0)),
                  pl.BlockSpec((rb, _K, cs), lambda i: (i, 0, 0)),
                  pl.BlockSpec((c, co), lambda i: (0, 0)),
                  pl.BlockSpec((c, co), lambda i: (0, 0)),
                  pl.BlockSpec((8, co), lambda i: (0, 0))],
        out_specs=pl.BlockSpec((rb, co), lambda i: (i, 0)),
        out_shape=jax.ShapeDtypeStruct((m, co), jnp.float32),
    )(xf, xs3, wt, wb, bias)


def _sc_gather_wsum(table, idx, w, kg, cu):
    e = idx.shape[0]
    v, c = table.shape
    nodes = e // kg
    info = plsc.get_sparse_core_info()
    nw = info.num_cores * info.num_subcores
    npw = nodes // nw
    ch = 8
    nchunk = npw // ch
    mesh = plsc.VectorSubcoreMesh(core_axis_name="c", subcore_axis_name="s")

    @functools.partial(
        pl.kernel, mesh=mesh,
        out_type=jax.ShapeDtypeStruct((nodes, cu), jnp.float32),
        scratch_types=[
            pltpu.VMEM((128,), jnp.int32),
            pltpu.VMEM((128,), jnp.int32),
            pltpu.VMEM((128, c), jnp.float32),
            pltpu.VMEM((128, c), jnp.float32),
            pltpu.VMEM((272,), jnp.float32),
            pltpu.VMEM((ch, cu), jnp.float32),
            pltpu.SemaphoreType.DMA,
        ])
    def k(table_hbm, idx_hbm, w_hbm, out_hbm,
          idx_a, idx_b, rows_a, rows_b, w_v, out_v, sem):
        wid = lax.axis_index("s") * info.num_cores + lax.axis_index("c")
        base_node = wid * npw

        def chunk_body(ci, _):
            nbase = base_node + ci * ch
            ebase = nbase * kg
            pltpu.sync_copy(idx_hbm.at[pl.ds(ebase, 128)], idx_a)
            pltpu.sync_copy(idx_hbm.at[pl.ds(ebase + 128, 128)], idx_b)
            pltpu.sync_copy(w_hbm.at[pl.ds(ebase, 256)],
                            w_v.at[pl.ds(0, 256)])
            pltpu.async_copy(table_hbm.at[idx_a], rows_a, sem).wait()
            pltpu.async_copy(table_hbm.at[idx_b], rows_b, sem).wait()

            for j in range(ch):
                rv = rows_a if j < 4 else rows_b
                rbase = (j % 4) * kg

                def ebody(ee, accs, rv=rv, rbase=rbase, j=j):
                    ws = w_v[pl.ds(j * kg + ee, 16)][0]
                    return tuple(
                        accs[cv] + rv[rbase + ee, pl.ds(cv * 16, 16)] * ws
                        for cv in range(cu // 16))

                accs = tuple(jnp.zeros((16,), jnp.float32)
                             for _ in range(cu // 16))
                accs = lax.fori_loop(0, kg, ebody, accs)
                for cv in range(cu // 16):
                    out_v[j, pl.ds(cv * 16, 16)] = accs[cv]

            pltpu.sync_copy(out_v, out_hbm.at[pl.ds(nbase, ch)])
            return 0

        lax.fori_loop(0, nchunk, chunk_body, 0)

    return k(table, idx, w)


def kernel(point_cloud, feat, params):
    p = params
    b, n, _ = point_cloud.shape
    m = b * n

    cov, sid1, gidx, wgeo = _knn_call(point_cloud)
    sid1_f = sid1.reshape(-1)
    gidx_f = gidx.reshape(-1)
    wgeo_f = wgeo.reshape(-1)

    nkey = jax.random.key(7)
    noise1 = jax.random.normal(jax.random.fold_in(nkey, 1), (b, 3, n),
                               jnp.float32) * 0.01
    noise2 = jax.random.normal(jax.random.fold_in(nkey, 2), (b, 16, n),
                               jnp.float32) * 0.01
    pc_rows = point_cloud.reshape(m, 3)
    n1r = noise1.transpose(0, 2, 1).reshape(m, 3)
    n2r = noise2.transpose(0, 2, 1).reshape(m, 16)
    ftr = feat.transpose(0, 2, 1).reshape(m, 64)
    h0 = jnp.concatenate([pc_rows, cov.reshape(m, 9), n1r], axis=1)

    w10, w11, w12 = p['c1_0_w'].T, p['c1_1_w'].T, p['c1_2_w'].T
    wpp1, wpp2 = p['pp1_w'], p['pp2_w']
    g1 = jnp.concatenate(
        [p['g1_dir'].transpose(1, 0, 2),
         jnp.zeros((64, 6, 64), jnp.float32)], axis=2).reshape(64, 768)
    g2 = jnp.concatenate(
        [p['g2_dir'].transpose(1, 0, 2),
         jnp.zeros((128, 6, 64), jnp.float32)], axis=2).reshape(128, 768)

    (y1, st1) = _dense([h0], [w10, _brow(p['c1_0_b'])], [64], [64],
                       lambda r, f: (lambda y: ([y], [y]))(
                           _mm(r[0], f[0]) + f[1][0:1, :]))
    aff1 = _affine_from_stats(st1, p['c1_0_g'], p['c1_0_be'], m)

    (y2, st2) = _dense([y1], [aff1, w11, _brow(p['c1_1_b'])], [64], [64],
                       lambda r, f: (lambda y: ([y], [y]))(
                           _mm(_aff_leaky(r[0], f[0]), f[1]) + f[2][0:1, :]))
    aff2 = _affine_from_stats(st2, p['c1_1_g'], p['c1_1_be'], m)

    (y3, st3) = _dense([y2], [aff2, w12, _brow(p['c1_2_b'])], [64], [64],
                       lambda r, f: (lambda y: ([y], [y]))(
                           _mm(_aff_leaky(r[0], f[0]), f[1]) + f[2][0:1, :]))
    aff3 = _affine_from_stats(st3, p['c1_2_g'], p['c1_2_be'], m)

    def body4(r, f):
        l0 = _aff_leaky(r[0], f[0])
        l0p = jnp.concatenate([l0, jnp.zeros_like(l0)], axis=1)
        t1 = _mm(l0, f[1])
        return [l0, l0p, t1], []

    l0, l0p, t1 = _dense([y3], [aff3, g1], [64, 128, 768], [], body4)

    xs1 = _sc_gather(l0p, sid1_f).reshape(m, _K, 128)
    agg1 = _sc_gather_wsum(t1.reshape(m * 6, 128), gidx_f, wgeo_f, 32, 64)
    l1 = _pp_call(l0, xs1, wpp1[:64], wpp1[64:], _brow(p['pp1_b']))

    xs2 = _sc_gather(l1, sid1_f).reshape(m, _K, 128)
    l2 = _pp_call(l1, xs2, wpp2[:128], wpp2[128:], _brow(p['pp2_b']))

    def body6(r, f):
        out1 = _leaky(_mm(r[0], f[0]) + f[1][0:1, :]
                      + _mm(jnp.maximum(r[1], 0.0), f[2]) + f[3][0:1, :])
        yc2 = _mm(out1, f[4]) + f[5][0:1, :]
        return [out1, yc2], [yc2]

    out1, yc2, stc2 = _dense(
        [l0, agg1],
        [p['g1_s_w'], _brow(p['g1_s_b']), p['g1_o_w'], _brow(p['g1_o_b']),
         p['c2_w'].T, _brow(p['c2_b'])],
        [128, 128], [128], body6)
    affc2 = _affine_from_stats(stc2, p['c2_g'], p['c2_be'], m)

    def body7(r, f):
        l4 = _aff_leaky(r[0], f[0])
        return [l4, _mm(l4, f[1])], []

    l4, t2 = _dense([yc2], [affc2, g2], [128, 768], [], body7)

    agg2 = _sc_gather_wsum(t2.reshape(m * 6, 128), gidx_f, wgeo_f, 32, 64)

    def body8(r, f):
        out2 = _leaky(_mm(r[1], f[0]) + f[1][0:1, :]
                      + _mm(jnp.maximum(r[2], 0.0), f[2]) + f[3][0:1, :])
        yf0 = (_mm(r[3], f[4]) + _mm(out2, f[5]) + _mm(r[0], f[6])
               + f[7][0:1, :])
        return [yf0], [yf0]

    wf0 = p['fc_0_w'].T
    yf0, stf0 = _dense(
        [l2, l4, agg2, out1],
        [p['g2_s_w'], _brow(p['g2_s_b']), p['g2_o_w'], _brow(p['g2_o_b']),
         wf0[:128], wf0[128:384], wf0[384:], _brow(p['fc_0_b'])],
        [512], [512], body8)
    afff0 = _affine_from_stats(stf0, p['fc_0_g'], p['fc_0_be'], m)

    yf1, stf1 = _dense([yf0], [afff0, p['fc_1_w'].T, _brow(p['fc_1_b'])],
                       [256], [256],
                       lambda r, f: (lambda y: ([y], [y]))(
                           _mm(_aff_leaky(r[0], f[0]), f[1]) + f[2][0:1, :]))
    afff1 = _affine_from_stats(stf1, p['fc_1_g'], p['fc_1_be'], m)

    yf2, stf2 = _dense([yf1], [afff1, p['fc_2_w'].T, _brow(p['fc_2_b'])],
                       [128], [128],
                       lambda r, f: (lambda y: ([y], [y]))(
                           _mm(_aff_leaky(r[0], f[0]), f[1]) + f[2][0:1, :]))
    afff2 = _affine_from_stats(stf2, p['fc_2_g'], p['fc_2_be'], m)

    def body11(r, f):
        x = _aff_leaky(r[0], f[0])
        l0f = _mm(x, f[1]) + f[2][0:1, :]
        yz = _mm(l0f, f[3]) + _mm(r[1], f[4]) + f[5][0:1, :]
        yr = _mm(l0f, f[6]) + _mm(r[1], f[7]) + f[8][0:1, :]
        return [l0f, yz, yr], [yz, yr]

    wz, wr = p['uz_w'].T, p['ur_w'].T
    l0f, yz, yr, stz, str_ = _dense(
        [yf2, ftr],
        [afff2, p['fc_3_w'].T, _brow(p['fc_3_b']),
         wz[:64], wz[64:], _brow(p['uz_b']),
         wr[:64], wr[64:], _brow(p['ur_b'])],
        [64, 64, 64], [64, 64], body11)
    affz = _affine_from_stats(stz, p['uz_g'], p['uz_be'], m)
    affr = _affine_from_stats(str_, p['ur_g'], p['ur_be'], m)

    def body12(r, f):
        rr = _aff_sig(r[0], f[0])
        yh = _mm(r[1], f[1]) + _mm(rr * r[2], f[2]) + f[3][0:1, :]
        return [yh], [yh]

    wh = p['uh_w'].T
    yh, sth = _dense([yr, l0f, ftr],
                     [affr, wh[:64], wh[64:], _brow(p['uh_b'])],
                     [64], [64], body12)
    affh = _affine_from_stats(sth, p['uh_g'], p['uh_be'], m)

    def body13(r, f):
        z = _aff_sig(r[0], f[0])
        hh = jnp.maximum(r[1] * f[1][0:1, :] + f[1][1:2, :], 0.0)
        f2 = (1.0 - z) * r[2] + z * hh
        ys0 = _mm(f2, f[2]) + _mm(r[3], f[3]) + f[4][0:1, :]
        return [ys0], [ys0]

    wsh0 = p['sh_0_w'].T
    ys0, sts0 = _dense([yz, yh, l0f, n2r],
                       [affz, affh, wsh0[:64], wsh0[64:], _brow(p['sh_0_b'])],
                       [64], [64], body13)
    affs0 = _affine_from_stats(sts0, p['sh_0_g'], p['sh_0_be'], m)

    def body14(r, f):
        s = _aff_leaky(r[0], f[0])
        seg = _mm(s, f[1]) + f[2][0:1, :]
        return [seg + r[1], seg], []

    sega, segb = _dense([ys0, pc_rows],
                        [affs0, p['sh_1_w'].T, _brow(p['sh_1_b'])],
                        [3, 3], [], body14)

    seg = segb.reshape(b, n, 3).transpose(0, 2, 1)
    segx = sega.reshape(b, n, 3).transpose(0, 2, 1)
    return (segx, seg)


# speedup vs baseline: 4.1371x; 1.8114x over previous
import functools

import jax
import jax.numpy as jnp
from jax import lax
from jax.experimental import pallas as pl
from jax.experimental.pallas import tpu as pltpu
from jax.experimental.pallas import tpu_sc as plsc

_K = 8
_RB = 128
_RBM = 1024


def _knn_body(pcb_ref, pca_ref, pct_ref, cov_ref, sid1_ref, sid2_ref, w_ref):
    bi = pl.program_id(0)
    n = pca_ref.shape[1]
    pcb = pcb_ref[0]
    pca = pca_ref[0]
    pat = pct_ref[0]

    inner = -2.0 * lax.dot_general(pcb, pca, (((1,), (1,)), ((), ())),
                                   preferred_element_type=jnp.float32)
    xxb = jnp.sum(pcb * pcb, axis=1, keepdims=True)
    xxa = (pat[0:1, :] * pat[0:1, :] + pat[1:2, :] * pat[1:2, :]
           + pat[2:3, :] * pat[2:3, :])
    pd = (-xxb) - inner - xxa

    iota = lax.broadcasted_iota(jnp.int32, (_RB, n), 1)
    big = jnp.float32(1e30)

    msk = jnp.zeros((_RB, n), jnp.float32)
    near_idx = []
    nfx = []
    for _ in range(_K):
        pm = pd - msk
        v = jnp.max(pm, axis=1, keepdims=True)
        idx = jnp.min(jnp.where(pm == v, iota, n), axis=1, keepdims=True)
        oh = (iota == idx)
        msk = msk + oh.astype(jnp.float32) * big
        near_idx.append(idx)
        nfx.append(jnp.dot(oh.astype(jnp.float32), pca,
                           preferred_element_type=jnp.float32,
                           precision=lax.Precision.HIGHEST))

    mean = (nfx[0] + nfx[1] + nfx[2] + nfx[3]
            + nfx[4] + nfx[5] + nfx[6] + nfx[7]) * jnp.float32(0.125)
    ckb = [(fx - mean).astype(jnp.bfloat16).astype(jnp.float32) for fx in nfx]
    cov_cols = []
    for a in range(3):
        for bcol in range(3):
            acc = ckb[0][:, a:a + 1] * ckb[0][:, bcol:bcol + 1]
            for k in range(1, _K):
                acc = acc + ckb[k][:, a:a + 1] * ckb[k][:, bcol:bcol + 1]
            cov_cols.append(acc)
    cov_ref[0] = jnp.concatenate(cov_cols, axis=1)

    sid1_ref[0] = jnp.concatenate(near_idx, axis=1) + bi * n

    msk2 = jnp.zeros((_RB, n), jnp.float32)
    zcol = jnp.zeros((_RB, 1), jnp.float32)
    dis_l, fid_l, cos6_l = [], [], []
    for _ in range(_K):
        pm = pd + msk2
        v = jnp.min(pm, axis=1, keepdims=True)
        idx = jnp.min(jnp.where(pm == v, iota, n), axis=1, keepdims=True)
        oh = (iota == idx)
        msk2 = msk2 + oh.astype(jnp.float32) * big
        fx = jnp.dot(oh.astype(jnp.float32), pca,
                     preferred_element_type=jnp.float32,
                     precision=lax.Precision.HIGHEST)
        d = fx - pcb
        dis = jnp.maximum(jnp.sqrt(jnp.sum(d * d, axis=1, keepdims=True)),
                          jnp.float32(1e-16))
        pcos = jnp.cos(d / dis) ** 2
        c6 = []
        for j in range(3):
            cj = pcos[:, j:j + 1]
            pos = d[:, j:j + 1] > 0
            c6.append(jnp.where(pos, zcol, cj))
            c6.append(jnp.where(pos, cj, zcol))
        cos6_l.append(jnp.concatenate(c6, axis=1))
        fid_l.append(idx)
        dis_l.append(dis)

    pdr = jnp.concatenate(dis_l, axis=1)
    p_r = jnp.max(pdr, axis=1, keepdims=True) * jnp.float32(1.1)
    p_d = (p_r - pdr) ** 2
    wnorm = p_d / (jnp.sum(p_d, axis=1, keepdims=True) + jnp.float32(1e-16))

    w_cols = [wnorm[:, k:k + 1] * cos6_l[k] for k in range(_K)]
    w_ref[0] = jnp.concatenate(w_cols, axis=1)
    sid2_ref[0] = jnp.concatenate(fid_l, axis=1) + bi * n


def _knn_call(pc):
    b, n, _ = pc.shape
    grid = (b, n // _RB)
    return pl.pallas_call(
        _knn_body,
        grid=grid,
        in_specs=[
            pl.BlockSpec((1, _RB, 3), lambda bi, ri: (bi, ri, 0)),
            pl.BlockSpec((1, n, 3), lambda bi, ri: (bi, 0, 0)),
            pl.BlockSpec((1, 3, n), lambda bi, ri: (bi, 0, 0)),
        ],
        out_specs=[
            pl.BlockSpec((1, _RB, 9), lambda bi, ri: (bi, ri, 0)),
            pl.BlockSpec((1, _RB, _K), lambda bi, ri: (bi, ri, 0)),
            pl.BlockSpec((1, _RB, _K), lambda bi, ri: (bi, ri, 0)),
            pl.BlockSpec((1, _RB, 48), lambda bi, ri: (bi, ri, 0)),
        ],
        out_shape=[
            jax.ShapeDtypeStruct((b, n, 9), jnp.float32),
            jax.ShapeDtypeStruct((b, n, _K), jnp.int32),
            jax.ShapeDtypeStruct((b, n, _K), jnp.int32),
            jax.ShapeDtypeStruct((b, n, 48), jnp.float32),
        ],
    )(pc, pc, pc.transpose(0, 2, 1))


def _dense(row_ins, fulls, out_dims, stats_dims, body, rbm=_RBM):
    m = row_ins[0].shape[0]
    nb = m // rbm
    nri, nfu, nod = len(row_ins), len(fulls), len(out_dims)
    n_stats = len(stats_dims)

    def kern(*refs):
        i = pl.program_id(0)
        rvals = [refs[j][...] for j in range(nri)]
        fvals = [refs[nri + j][...] for j in range(nfu)]
        outs, stats = body(rvals, fvals)
        for j in range(nod):
            refs[nri + nfu + j][...] = outs[j]
        for j in range(n_stats):
            y = stats[j]
            c = y.shape[1]
            contrib = jnp.concatenate(
                [jnp.sum(y, axis=0, keepdims=True),
                 jnp.sum(y * y, axis=0, keepdims=True),
                 jnp.zeros((6, c), jnp.float32)], axis=0)
            ref = refs[nri + nfu + nod + j]

            @pl.when(i == 0)
            def _():
                ref[...] = contrib

            @pl.when(i > 0)
            def _():
                ref[...] += contrib

    in_specs = (
        [pl.BlockSpec((rbm, t.shape[1]), lambda i: (i, 0)) for t in row_ins] +
        [pl.BlockSpec(t.shape, (lambda nd: (lambda i: (0,) * nd))(t.ndim))
         for t in fulls])
    out_specs = (
        [pl.BlockSpec((rbm, c), lambda i: (i, 0)) for c in out_dims] +
        [pl.BlockSpec((8, c), lambda i: (0, 0)) for c in stats_dims])
    out_shape = (
        [jax.ShapeDtypeStruct((m, c), jnp.float32) for c in out_dims] +
        [jax.ShapeDtypeStruct((8, c), jnp.float32) for c in stats_dims])
    res = pl.pallas_call(
        kern, grid=(nb,), in_specs=in_specs, out_specs=out_specs,
        out_shape=out_shape,
    )(*row_ins, *fulls)
    return res


def _mm(x, w):
    return jnp.dot(x, w, preferred_element_type=jnp.float32)


def _leaky(x):
    return jnp.where(x >= 0, x, jnp.float32(0.2) * x)


def _aff_leaky(y, aff):
    return _leaky(y * aff[0:1, :] + aff[1:2, :])


def _aff_sig(y, aff):
    return jax.nn.sigmoid(y * aff[0:1, :] + aff[1:2, :])


def _affine_from_stats(stats, g, be, mtot):
    mean = stats[0] / mtot
    var = stats[1] / mtot - mean * mean
    sc = g / jnp.sqrt(var + 1e-5)
    sh = be - mean * sc
    return jnp.concatenate([sc[None, :], sh[None, :],
                            jnp.zeros((6, sc.shape[0]), jnp.float32)], axis=0)


def _brow(b):
    return jnp.concatenate([b[None, :], jnp.zeros((7, b.shape[0]),
                                                  jnp.float32)], axis=0)


def _sc_gather(table, idx):
    e = idx.shape[0]
    v, c = table.shape
    info = plsc.get_sparse_core_info()
    nw = info.num_cores * info.num_subcores
    epw = e // nw
    nchunk = epw // 128
    mesh = plsc.VectorSubcoreMesh(core_axis_name="c", subcore_axis_name="s")

    @functools.partial(
        pl.kernel, mesh=mesh,
        out_type=jax.ShapeDtypeStruct((e, c), jnp.float32),
        scratch_types=[
            pltpu.VMEM((128,), jnp.int32),
            pltpu.VMEM((128, c), jnp.float32),
            pltpu.SemaphoreType.DMA,
        ])
    def k(table_hbm, idx_hbm, out_hbm, idx_v, rows_v, sem):
        wid = lax.axis_index("s") * info.num_cores + lax.axis_index("c")
        base_e = wid * epw

        def chunk_body(ci, _):
            ebase = base_e + ci * 128
            pltpu.sync_copy(idx_hbm.at[pl.ds(ebase, 128)], idx_v)
            pltpu.async_copy(table_hbm.at[idx_v], rows_v, sem).wait()
            pltpu.sync_copy(rows_v, out_hbm.at[pl.ds(ebase, 128)])
            return 0

        lax.fori_loop(0, nchunk, chunk_body, 0)

    return k(table, idx)


def _pp_call(xf, xs3, wt, wb, bias):
    m, c = xf.shape
    cs = xs3.shape[2]
    co = wt.shape[1]
    rb = 128

    def body(xf_ref, xs_ref, wt_ref, wb_ref, b_ref, o_ref):
        xt = xf_ref[...]
        base = jnp.dot(xt, wt_ref[...],
                       preferred_element_type=jnp.float32) + b_ref[0:1, :]
        acc = None
        for k in range(_K):
            diff = xs_ref[:, k, :c] - xt
            h = jnp.maximum(
                base + jnp.dot(diff, wb_ref[...],
                               preferred_element_type=jnp.float32), 0.0)
            acc = h if acc is None else jnp.maximum(acc, h)
        o_ref[...] = acc

    return pl.pallas_call(
        body, grid=(m // rb,),
        in_specs=[pl.BlockSpec((rb, c), lambda i: (i, 0)),
                  pl.BlockSpec((rb, _K, cs), lambda i: (i, 0, 0)),
                  pl.BlockSpec((c, co), lambda i: (0, 0)),
                  pl.BlockSpec((c, co), lambda i: (0, 0)),
                  pl.BlockSpec((8, co), lambda i: (0, 0))],
        out_specs=pl.BlockSpec((rb, co), lambda i: (i, 0)),
        out_shape=jax.ShapeDtypeStruct((m, co), jnp.float32),
    )(xf, xs3, wt, wb, bias)


def _geo_agg_call(xr, w6):
    m = xr.shape[0]
    rb = 256

    def body(xr_ref, w6_ref, o_ref):
        acc = jnp.zeros((rb, 64), jnp.float32)
        for k in range(_K):
            for dd in range(6):
                acc = acc + (xr_ref[:, k, dd * 64:(dd + 1) * 64]
                             * w6_ref[:, k * 6 + dd:k * 6 + dd + 1])
        o_ref[...] = acc

    return pl.pallas_call(
        body, grid=(m // rb,),
        in_specs=[pl.BlockSpec((rb, _K, 384), lambda i: (i, 0, 0)),
                  pl.BlockSpec((rb, 48), lambda i: (i, 0))],
        out_specs=pl.BlockSpec((rb, 64), lambda i: (i, 0)),
        out_shape=jax.ShapeDtypeStruct((m, 64), jnp.float32),
    )(xr, w6)


def kernel(point_cloud, feat, params):
    p = params
    b, n, _ = point_cloud.shape
    m = b * n

    cov, sid1, sid2, w6 = _knn_call(point_cloud)
    sid1_f = sid1.reshape(-1)
    sid2_f = sid2.reshape(-1)
    w6r = w6.reshape(m, 48)

    nkey = jax.random.key(7)
    noise1 = jax.random.normal(jax.random.fold_in(nkey, 1), (b, 3, n),
                               jnp.float32) * 0.01
    noise2 = jax.random.normal(jax.random.fold_in(nkey, 2), (b, 16, n),
                               jnp.float32) * 0.01
    pc_rows = point_cloud.reshape(m, 3)
    n1r = noise1.transpose(0, 2, 1).reshape(m, 3)
    n2r = noise2.transpose(0, 2, 1).reshape(m, 16)
    ftr = feat.transpose(0, 2, 1).reshape(m, 64)
    h0 = jnp.concatenate([pc_rows, cov.reshape(m, 9), n1r], axis=1)

    w10, w11, w12 = p['c1_0_w'].T, p['c1_1_w'].T, p['c1_2_w'].T
    wpp1, wpp2 = p['pp1_w'], p['pp2_w']
    g1 = p['g1_dir'].transpose(1, 0, 2).reshape(64, 384)
    g2 = p['g2_dir'].transpose(1, 0, 2).reshape(128, 384)

    (y1, st1) = _dense([h0], [w10, _brow(p['c1_0_b'])], [64], [64],
                       lambda r, f: (lambda y: ([y], [y]))(
                           _mm(r[0], f[0]) + f[1][0:1, :]))
    aff1 = _affine_from_stats(st1, p['c1_0_g'], p['c1_0_be'], m)

    (y2, st2) = _dense([y1], [aff1, w11, _brow(p['c1_1_b'])], [64], [64],
                       lambda r, f: (lambda y: ([y], [y]))(
                           _mm(_aff_leaky(r[0], f[0]), f[1]) + f[2][0:1, :]))
    aff2 = _affine_from_stats(st2, p['c1_1_g'], p['c1_1_be'], m)

    (y3, st3) = _dense([y2], [aff2, w12, _brow(p['c1_2_b'])], [64], [64],
                       lambda r, f: (lambda y: ([y], [y]))(
                           _mm(_aff_leaky(r[0], f[0]), f[1]) + f[2][0:1, :]))
    aff3 = _affine_from_stats(st3, p['c1_2_g'], p['c1_2_be'], m)

    def body4(r, f):
        l0 = _aff_leaky(r[0], f[0])
        l0p = jnp.concatenate([l0, jnp.zeros_like(l0)], axis=1)
        t1 = _mm(l0, f[1])
        return [l0, l0p, t1], []

    l0, l0p, t1 = _dense([y3], [aff3, g1], [64, 128, 384], [], body4)

    xs1 = _sc_gather(l0p, sid1_f).reshape(m, _K, 128)
    xr1 = _sc_gather(t1, sid2_f).reshape(m, _K, 384)
    agg1 = _geo_agg_call(xr1, w6r)
    l1 = _pp_call(l0, xs1, wpp1[:64], wpp1[64:], _brow(p['pp1_b']))

    xs2 = _sc_gather(l1, sid1_f).reshape(m, _K, 128)
    l2 = _pp_call(l1, xs2, wpp2[:128], wpp2[128:], _brow(p['pp2_b']))

    def body6(r, f):
        out1 = _leaky(_mm(r[0], f[0]) + f[1][0:1, :]
                      + _mm(jnp.maximum(r[1], 0.0), f[2]) + f[3][0:1, :])
        yc2 = _mm(out1, f[4]) + f[5][0:1, :]
        return [out1, yc2], [yc2]

    out1, yc2, stc2 = _dense(
        [l0, agg1],
        [p['g1_s_w'], _brow(p['g1_s_b']), p['g1_o_w'], _brow(p['g1_o_b']),
         p['c2_w'].T, _brow(p['c2_b'])],
        [128, 128], [128], body6)
    affc2 = _affine_from_stats(stc2, p['c2_g'], p['c2_be'], m)

    def body7(r, f):
        l4 = _aff_leaky(r[0], f[0])
        return [l4, _mm(l4, f[1])], []

    l4, t2 = _dense([yc2], [affc2, g2], [128, 384], [], body7)

    xr2 = _sc_gather(t2, sid2_f).reshape(m, _K, 384)
    agg2 = _geo_agg_call(xr2, w6r)

    def body8(r, f):
        out2 = _leaky(_mm(r[1], f[0]) + f[1][0:1, :]
                      + _mm(jnp.maximum(r[2], 0.0), f[2]) + f[3][0:1, :])
        yf0 = (_mm(r[3], f[4]) + _mm(out2, f[5]) + _mm(r[0], f[6])
               + f[7][0:1, :])
        return [yf0], [yf0]

    wf0 = p['fc_0_w'].T
    yf0, stf0 = _dense(
        [l2, l4, agg2, out1],
        [p['g2_s_w'], _brow(p['g2_s_b']), p['g2_o_w'], _brow(p['g2_o_b']),
         wf0[:128], wf0[128:384], wf0[384:], _brow(p['fc_0_b'])],
        [512], [512], body8)
    afff0 = _affine_from_stats(stf0, p['fc_0_g'], p['fc_0_be'], m)

    yf1, stf1 = _dense([yf0], [afff0, p['fc_1_w'].T, _brow(p['fc_1_b'])],
                       [256], [256],
                       lambda r, f: (lambda y: ([y], [y]))(
                           _mm(_aff_leaky(r[0], f[0]), f[1]) + f[2][0:1, :]))
    afff1 = _affine_from_stats(stf1, p['fc_1_g'], p['fc_1_be'], m)

    yf2, stf2 = _dense([yf1], [afff1, p['fc_2_w'].T, _brow(p['fc_2_b'])],
                       [128], [128],
                       lambda r, f: (lambda y: ([y], [y]))(
                           _mm(_aff_leaky(r[0], f[0]), f[1]) + f[2][0:1, :]))
    afff2 = _affine_from_stats(stf2, p['fc_2_g'], p['fc_2_be'], m)

    def body11(r, f):
        x = _aff_leaky(r[0], f[0])
        l0f = _mm(x, f[1]) + f[2][0:1, :]
        yz = _mm(l0f, f[3]) + _mm(r[1], f[4]) + f[5][0:1, :]
        yr = _mm(l0f, f[6]) + _mm(r[1], f[7]) + f[8][0:1, :]
        return [l0f, yz, yr], [yz, yr]

    wz, wr = p['uz_w'].T, p['ur_w'].T
    l0f, yz, yr, stz, str_ = _dense(
        [yf2, ftr],
        [afff2, p['fc_3_w'].T, _brow(p['fc_3_b']),
         wz[:64], wz[64:], _brow(p['uz_b']),
         wr[:64], wr[64:], _brow(p['ur_b'])],
        [64, 64, 64], [64, 64], body11)
    affz = _affine_from_stats(stz, p['uz_g'], p['uz_be'], m)
    affr = _affine_from_stats(str_, p['ur_g'], p['ur_be'], m)

    def body12(r, f):
        rr = _aff_sig(r[0], f[0])
        yh = _mm(r[1], f[1]) + _mm(rr * r[2], f[2]) + f[3][0:1, :]
        return [yh], [yh]

    wh = p['uh_w'].T
    yh, sth = _dense([yr, l0f, ftr],
                     [affr, wh[:64], wh[64:], _brow(p['uh_b'])],
                     [64], [64], body12)
    affh = _affine_from_stats(sth, p['uh_g'], p['uh_be'], m)

    def body13(r, f):
        z = _aff_sig(r[0], f[0])
        hh = jnp.maximum(r[1] * f[1][0:1, :] + f[1][1:2, :], 0.0)
        f2 = (1.0 - z) * r[2] + z * hh
        ys0 = _mm(f2, f[2]) + _mm(r[3], f[3]) + f[4][0:1, :]
        return [ys0], [ys0]

    wsh0 = p['sh_0_w'].T
    ys0, sts0 = _dense([yz, yh, l0f, n2r],
                       [affz, affh, wsh0[:64], wsh0[64:], _brow(p['sh_0_b'])],
                       [64], [64], body13)
    affs0 = _affine_from_stats(sts0, p['sh_0_g'], p['sh_0_be'], m)

    def body14(r, f):
        s = _aff_leaky(r[0], f[0])
        seg = _mm(s, f[1]) + f[2][0:1, :]
        return [seg + r[1], seg], []

    sega, segb = _dense([ys0, pc_rows],
                        [affs0, p['sh_1_w'].T, _brow(p['sh_1_b'])],
                        [3, 3], [], body14)

    seg = segb.reshape(b, n, 3).transpose(0, 2, 1)
    segx = sega.reshape(b, n, 3).transpose(0, 2, 1)
    return (segx, seg)
